# baseline (speedup 1.0000x reference)
import functools

import jax
import jax.numpy as jnp
from jax import lax
from jax.experimental import pallas as pl
from jax.experimental.pallas import tpu as pltpu
from jax.experimental.pallas import tpu_sc as plsc

D = 64
B = 16384
NC = 2
NS = 16
NW = NC * NS
BPW = B // NW
CHUNK = 128
NCHUNK = BPW // CHUNK


def _gather_sc(idx3, table):
    mesh = plsc.VectorSubcoreMesh(core_axis_name="c", subcore_axis_name="s")

    @functools.partial(
        pl.kernel,
        mesh=mesh,
        out_type=jax.ShapeDtypeStruct((B, D), jnp.float32),
        scratch_types=[
            pltpu.VMEM((NCHUNK, CHUNK), jnp.int32),
            pltpu.VMEM((BPW, D), jnp.float32),
            pltpu.SemaphoreType.DMA,
        ],
        compiler_params=pltpu.CompilerParams(use_tc_tiling_on_sc=False),
    )
    def k(idx_hbm, table_hbm, out_hbm, idx_v, rows_v, sem):
        wid = lax.axis_index("s") * NC + lax.axis_index("c")
        base = wid * BPW
        pltpu.sync_copy(idx_hbm.at[wid], idx_v)
        copies = [
            pltpu.async_copy(
                table_hbm.at[idx_v.at[j]],
                rows_v.at[pl.ds(j * CHUNK, CHUNK)],
                sem,
            )
            for j in range(NCHUNK)
        ]
        for c in copies:
            c.wait()
        pltpu.sync_copy(rows_v, out_hbm.at[pl.ds(base, BPW)])

    return k(idx3, table)


def _bn_tc(e, w, b):
    def body(e_ref, w_ref, b_ref, o_ref):
        x = e_ref[...]
        mean = jnp.mean(x, axis=0, keepdims=True)
        xc = x - mean
        var = jnp.mean(xc * xc, axis=0, keepdims=True)
        inv = lax.rsqrt(var + 1e-5)
        o_ref[...] = xc * (inv * w_ref[...]) + b_ref[...]

    return pl.pallas_call(
        body,
        out_shape=jax.ShapeDtypeStruct((B, D), jnp.float32),
    )(e, w.reshape(1, D), b.reshape(1, D))


@jax.jit
def kernel(indices, embed_weight, bn_weight, bn_bias):
    idx3 = indices.astype(jnp.int32).reshape(NW, NCHUNK, CHUNK)
    e = _gather_sc(idx3, embed_weight)
    return _bn_tc(e, bn_weight, bn_bias)


# speedup vs baseline: 2.3959x; 2.3959x over previous
import functools

import jax
import jax.numpy as jnp
from jax import lax
from jax.experimental import pallas as pl
from jax.experimental.pallas import tpu as pltpu
from jax.experimental.pallas import tpu_sc as plsc

D = 64
B = 16384
NC = 2
NS = 16
NW = NC * NS
BPW = B // NW


def _gather_sc(idx2, table3):
    mesh = plsc.VectorSubcoreMesh(core_axis_name="c", subcore_axis_name="s")

    @functools.partial(
        pl.kernel,
        mesh=mesh,
        out_type=jax.ShapeDtypeStruct((B, D), jnp.float32),
        scratch_types=[
            pltpu.VMEM((BPW,), jnp.int32),
            pltpu.VMEM((BPW, D), jnp.float32),
            pltpu.SemaphoreType.DMA,
        ],
    )
    def k(idx_hbm, table_hbm, out_hbm, idx_v, rows, semg):
        wid = lax.axis_index("s") * NC + lax.axis_index("c")
        base = wid * BPW
        pltpu.sync_copy(idx_hbm.at[wid], idx_v)
        copies = []
        for g in range(BPW // 16):
            rv = idx_v[pl.ds(g * 16, 16)]
            tv = lax.shift_right_logical(rv, 3)
            sv = rv & 7
            for l in range(16):
                copies.append(
                    pltpu.async_copy(
                        table_hbm.at[tv[l], sv[l]],
                        rows.at[g * 16 + l], semg))
        for c in copies:
            c.wait()
        pltpu.sync_copy(rows, out_hbm.at[pl.ds(base, BPW)])

    return k(idx2, table3)


def _bn_tc(e, w, b):
    def body(e_ref, w_ref, b_ref, o_ref):
        x = e_ref[...]
        mean = jnp.mean(x, axis=0, keepdims=True)
        xc = x - mean
        var = jnp.mean(xc * xc, axis=0, keepdims=True)
        inv = lax.rsqrt(var + 1e-5)
        o_ref[...] = xc * (inv * w_ref[...]) + b_ref[...]

    return pl.pallas_call(
        body,
        out_shape=jax.ShapeDtypeStruct((B, D), jnp.float32),
    )(e, w.reshape(1, D), b.reshape(1, D))


@jax.jit
def kernel(indices, embed_weight, bn_weight, bn_bias):
    idx2 = indices.astype(jnp.int32).reshape(NW, BPW)
    table3 = embed_weight.reshape(-1, 8, D)
    e = _gather_sc(idx2, table3)
    return _bn_tc(e, bn_weight, bn_bias)
